# Initial kernel scaffold; baseline (speedup 1.0000x reference)
import jax
import jax.numpy as jnp
from jax.experimental import pallas as pl


def kernel(x, edge_index, W1, b1, W2, b2):
    raise NotImplementedError("write your pallas kernel here")


# speedup vs baseline: 12.7344x; 12.7344x over previous
import functools

import jax
import jax.numpy as jnp
from jax import lax
from jax.experimental import pallas as pl
from jax.experimental.pallas import tpu as pltpu
from jax.experimental.pallas import tpu_sc as plsc

N_NODES = 10000
N_EDGES = 320000
D_IN = 128
D_HID = 128
D_OUT = 64

NPAD = 10240
N_TILES = 32
EDGES_PER_TILE = N_EDGES // N_TILES
CHUNK = 128
N_CHUNKS = -(-EDGES_PER_TILE // CHUNK)
EDGES_PER_TILE_PAD = N_CHUNKS * CHUNK
ROWS_PER_TILE = NPAD // 16
DEG_W = 16

_MESH = dict(core_axis_name="c", subcore_axis_name="s")


def _zero_fill(buf, rows, width):
    @pl.loop(0, rows)
    def _(r):
        @pl.loop(0, width, step=16)
        def _(cc):
            buf[r, pl.ds(cc, 16)] = jnp.zeros((16,), jnp.float32)


def _deg_kernel(dst_hbm, out_hbm, acc, dstbuf, ones, zbuf):
    c = lax.axis_index("c")
    s = lax.axis_index("s")

    @pl.loop(0, CHUNK)
    def _(r):
        ones[r, :] = jnp.full((16,), 1.0, jnp.float32)

    _zero_fill(zbuf, CHUNK, DEG_W)
    @pl.loop(0, ROWS_PER_TILE, step=CHUNK)
    def _(k):
        pltpu.sync_copy(zbuf, acc.at[pl.ds(s * ROWS_PER_TILE + k, CHUNK)])

    plsc.subcore_barrier()

    pltpu.sync_copy(dst_hbm.at[c, s], dstbuf)

    @pl.loop(0, N_CHUNKS)
    def _(j):
        pltpu.sync_copy(ones, acc.at[dstbuf.at[j]], add=True)

    plsc.subcore_barrier()
    pltpu.sync_copy(acc.at[pl.ds(s * ROWS_PER_TILE, ROWS_PER_TILE)],
                    out_hbm.at[c, pl.ds(s * ROWS_PER_TILE, ROWS_PER_TILE)])


def _make_deg(dst_e):
    return pl.kernel(
        _deg_kernel,
        out_type=jax.ShapeDtypeStruct((2, NPAD, DEG_W), jnp.float32),
        mesh=plsc.VectorSubcoreMesh(**_MESH),
        scratch_types=[
            pltpu.VMEM_SHARED((NPAD, DEG_W), jnp.float32),
            pltpu.VMEM((N_CHUNKS, CHUNK), jnp.int32),
            pltpu.VMEM((CHUNK, DEG_W), jnp.float32),
            pltpu.VMEM((CHUNK, DEG_W), jnp.float32),
        ],
    )(dst_e)


def _agg_kernel(feat, src_hbm, dst_hbm, g_hbm, out_hbm,
                acc, srcbuf, dstbuf, rows, sem):
    c = lax.axis_index("c")
    s = lax.axis_index("s")

    _zero_fill(rows, CHUNK, feat)
    @pl.loop(0, ROWS_PER_TILE, step=CHUNK)
    def _(k):
        pltpu.sync_copy(rows, acc.at[pl.ds(s * ROWS_PER_TILE + k, CHUNK)])

    plsc.subcore_barrier()

    pltpu.sync_copy(src_hbm.at[c, s], srcbuf)
    pltpu.sync_copy(dst_hbm.at[c, s], dstbuf)

    @pl.loop(0, N_CHUNKS)
    def _(j):
        pltpu.async_copy(g_hbm.at[srcbuf.at[j]], rows, sem).wait()
        pltpu.sync_copy(rows, acc.at[dstbuf.at[j]], add=True)

    plsc.subcore_barrier()
    pltpu.sync_copy(acc.at[pl.ds(s * ROWS_PER_TILE, ROWS_PER_TILE)],
                    out_hbm.at[c, pl.ds(s * ROWS_PER_TILE, ROWS_PER_TILE)])


def _make_agg(feat, src_e, dst_e, g):
    return pl.kernel(
        functools.partial(_agg_kernel, feat),
        out_type=jax.ShapeDtypeStruct((2, NPAD, feat), jnp.float32),
        mesh=plsc.VectorSubcoreMesh(**_MESH),
        scratch_types=[
            pltpu.VMEM_SHARED((NPAD, feat), jnp.float32),
            pltpu.VMEM((N_CHUNKS, CHUNK), jnp.int32),
            pltpu.VMEM((N_CHUNKS, CHUNK), jnp.int32),
            pltpu.VMEM((CHUNK, feat), jnp.float32),
            pltpu.SemaphoreType.DMA,
        ],
    )(src_e, dst_e, g)


ROW_BLK = 256
_GRID = NPAD // ROW_BLK


def _dinv_blk(degA_ref, degB_ref):
    deg = degA_ref[:, 0:1] + degB_ref[:, 0:1] + 1.0
    return lax.rsqrt(deg)


def _mm1_body(x_ref, w_ref, o_ref):
    o_ref[...] = jnp.dot(x_ref[...], w_ref[...],
                         precision=lax.Precision.HIGHEST,
                         preferred_element_type=jnp.float32)


def _g1_body(degA_ref, degB_ref, h_ref, o_ref):
    o_ref[...] = _dinv_blk(degA_ref, degB_ref) * h_ref[...]


def _mid_body(degA_ref, degB_ref, rA_ref, rB_ref, g1_ref, b1_ref, w2_ref,
              o_ref):
    dinv = _dinv_blk(degA_ref, degB_ref)
    z = jax.nn.relu(dinv * (rA_ref[...] + rB_ref[...] + g1_ref[...])
                    + b1_ref[...])
    h2 = jnp.dot(z, w2_ref[...], precision=lax.Precision.HIGHEST,
                 preferred_element_type=jnp.float32)
    o_ref[...] = dinv * h2


def _out_body(degA_ref, degB_ref, rA_ref, rB_ref, g2_ref, b2_ref, o_ref):
    dinv = _dinv_blk(degA_ref, degB_ref)
    o_ref[...] = dinv * (rA_ref[...] + rB_ref[...] + g2_ref[...]) + b2_ref[...]


def _row_spec(width):
    return pl.BlockSpec((ROW_BLK, width), lambda i: (i, 0))


def _full_spec(shape):
    return pl.BlockSpec(shape, lambda i: tuple(0 for _ in shape))


def kernel(x, edge_index, W1, b1, W2, b2):
    src = edge_index[0].astype(jnp.int32)
    dst = edge_index[1].astype(jnp.int32)
    pad = EDGES_PER_TILE_PAD - EDGES_PER_TILE
    src_e = jnp.pad(src.reshape(N_TILES, EDGES_PER_TILE), ((0, 0), (0, pad)),
                    constant_values=0).reshape(2, 16, N_CHUNKS, CHUNK)
    dst_e = jnp.pad(dst.reshape(N_TILES, EDGES_PER_TILE), ((0, 0), (0, pad)),
                    constant_values=N_NODES).reshape(2, 16, N_CHUNKS, CHUNK)
    x_pad = jnp.pad(x, ((0, NPAD - N_NODES), (0, 0)))

    deg = _make_deg(dst_e)
    degA, degB = deg[0], deg[1]

    h1 = pl.pallas_call(
        _mm1_body,
        grid=(_GRID,),
        in_specs=[_row_spec(D_IN), _full_spec((D_IN, D_HID))],
        out_specs=_row_spec(D_HID),
        out_shape=jax.ShapeDtypeStruct((NPAD, D_HID), jnp.float32),
    )(x_pad, W1)

    g1 = pl.pallas_call(
        _g1_body,
        grid=(_GRID,),
        in_specs=[_row_spec(DEG_W), _row_spec(DEG_W), _row_spec(D_HID)],
        out_specs=_row_spec(D_HID),
        out_shape=jax.ShapeDtypeStruct((NPAD, D_HID), jnp.float32),
    )(degA, degB, h1)

    r1 = _make_agg(D_HID, src_e, dst_e, g1)

    W2p = jnp.pad(W2, ((0, 0), (0, D_HID - D_OUT)))
    b2p = jnp.pad(b2, (0, D_HID - D_OUT))

    g2 = pl.pallas_call(
        _mid_body,
        grid=(_GRID,),
        in_specs=[_row_spec(DEG_W), _row_spec(DEG_W), _row_spec(D_HID),
                  _row_spec(D_HID), _row_spec(D_HID),
                  _full_spec((1, D_HID)), _full_spec((D_HID, D_HID))],
        out_specs=_row_spec(D_HID),
        out_shape=jax.ShapeDtypeStruct((NPAD, D_HID), jnp.float32),
    )(degA, degB, r1[0], r1[1], g1, b1.reshape(1, D_HID), W2p)

    r2 = _make_agg(D_HID, src_e, dst_e, g2)

    out = pl.pallas_call(
        _out_body,
        grid=(_GRID,),
        in_specs=[_row_spec(DEG_W), _row_spec(DEG_W), _row_spec(D_HID),
                  _row_spec(D_HID), _row_spec(D_HID), _full_spec((1, D_HID))],
        out_specs=_row_spec(D_HID),
        out_shape=jax.ShapeDtypeStruct((NPAD, D_HID), jnp.float32),
    )(degA, degB, r2[0], r2[1], g2, b2p.reshape(1, D_HID))

    return out[:N_NODES, :D_OUT]


# speedup vs baseline: 12.7399x; 1.0004x over previous
import functools

import jax
import jax.numpy as jnp
from jax import lax
from jax.experimental import pallas as pl
from jax.experimental.pallas import tpu as pltpu
from jax.experimental.pallas import tpu_sc as plsc

N_NODES = 10000
N_EDGES = 320000
D_IN = 128
D_HID = 128
D_OUT = 64

NPAD = 10240
ACC_N = 10240
N_TILES = 32
EDGES_PER_TILE = N_EDGES // N_TILES
CHUNK = 128
N_CHUNKS = 79
ROWS_PER_TILE = ACC_N // 16
DEG_W = 16

_MESH = dict(core_axis_name="c", subcore_axis_name="s")


def _zero_fill(buf, rows, width):
    @pl.loop(0, rows)
    def _(r):
        @pl.loop(0, width, step=16)
        def _(cc):
            buf[r, pl.ds(cc, 16)] = jnp.zeros((16,), jnp.float32)


def _zero_acc(acc, zbuf, s):
    base = s * ROWS_PER_TILE

    @pl.loop(0, 512, step=CHUNK)
    def _(k):
        pltpu.sync_copy(zbuf, acc.at[pl.ds(base + k, CHUNK)])

    pltpu.sync_copy(zbuf.at[pl.ds(0, ROWS_PER_TILE - 512)],
                    acc.at[pl.ds(base + 512, ROWS_PER_TILE - 512)])


def _readout(acc, out_hbm, c, s):
    pltpu.sync_copy(acc.at[pl.ds(s * ROWS_PER_TILE, ROWS_PER_TILE)],
                    out_hbm.at[c, pl.ds(s * ROWS_PER_TILE, ROWS_PER_TILE)])


def _deg_kernel(dst_hbm, out_hbm, acc, dstbuf, ones, zbuf):
    c = lax.axis_index("c")
    s = lax.axis_index("s")

    @pl.loop(0, CHUNK)
    def _(r):
        ones[r, :] = jnp.full((16,), 1.0, jnp.float32)

    _zero_fill(zbuf, CHUNK, DEG_W)
    _zero_acc(acc, zbuf, s)
    plsc.subcore_barrier()

    pltpu.sync_copy(dst_hbm.at[c, s], dstbuf)

    @pl.loop(0, N_CHUNKS)
    def _(j):
        pltpu.sync_copy(ones, acc.at[dstbuf.at[j]], add=True)

    plsc.subcore_barrier()
    _readout(acc, out_hbm, c, s)


def _make_deg(dst_e):
    return pl.kernel(
        _deg_kernel,
        out_type=jax.ShapeDtypeStruct((2, NPAD, DEG_W), jnp.float32),
        mesh=plsc.VectorSubcoreMesh(**_MESH),
        scratch_types=[
            pltpu.VMEM_SHARED((ACC_N, DEG_W), jnp.float32),
            pltpu.VMEM((N_CHUNKS, CHUNK), jnp.int32),
            pltpu.VMEM((CHUNK, DEG_W), jnp.float32),
            pltpu.VMEM((CHUNK, DEG_W), jnp.float32),
        ],
    )(dst_e)


def _agg_kernel(src_hbm, dst_hbm, g_hbm, out_hbm, acc,
                srcbuf, dstbuf, rows, sem):
    c = lax.axis_index("c")
    s = lax.axis_index("s")

    _zero_fill(rows, CHUNK, D_HID)
    _zero_acc(acc, rows, s)
    plsc.subcore_barrier()

    pltpu.sync_copy(src_hbm.at[c, s], srcbuf)
    pltpu.sync_copy(dst_hbm.at[c, s], dstbuf)

    @pl.loop(0, N_CHUNKS)
    def _(j):
        pltpu.async_copy(g_hbm.at[srcbuf.at[j]], rows, sem).wait()
        pltpu.sync_copy(rows, acc.at[dstbuf.at[j]], add=True)

    plsc.subcore_barrier()
    _readout(acc, out_hbm, c, s)


def _make_agg(src_e, dst_e, g):
    return pl.kernel(
        _agg_kernel,
        out_type=jax.ShapeDtypeStruct((2, NPAD, D_HID), jnp.float32),
        mesh=plsc.VectorSubcoreMesh(**_MESH),
        scratch_types=[
            pltpu.VMEM_SHARED((ACC_N, D_HID), jnp.float32),
            pltpu.VMEM((N_CHUNKS, CHUNK), jnp.int32),
            pltpu.VMEM((N_CHUNKS, CHUNK), jnp.int32),
            pltpu.VMEM((CHUNK, D_HID), jnp.float32),
            pltpu.SemaphoreType.DMA,
        ],
    )(src_e, dst_e, g)


ROW_BLK = 256
_GRID = NPAD // ROW_BLK


def _dinv_blk(degA_ref, degB_ref):
    deg = degA_ref[:, 0:1] + degB_ref[:, 0:1] + 1.0
    return lax.rsqrt(deg)


def _mm1_body(x_ref, w_ref, o_ref):
    o_ref[...] = jnp.dot(x_ref[...], w_ref[...],
                         precision=lax.Precision.HIGHEST,
                         preferred_element_type=jnp.float32)


def _g1_body(degA_ref, degB_ref, h_ref, o_ref):
    o_ref[...] = _dinv_blk(degA_ref, degB_ref) * h_ref[...]


def _mid_body(degA_ref, degB_ref, rA_ref, rB_ref, g1_ref, b1_ref, w2_ref,
              o_ref):
    dinv = _dinv_blk(degA_ref, degB_ref)
    z = jax.nn.relu(dinv * (rA_ref[...] + rB_ref[...] + g1_ref[...])
                    + b1_ref[...])
    h2 = jnp.dot(z, w2_ref[...], precision=lax.Precision.HIGHEST,
                 preferred_element_type=jnp.float32)
    o_ref[...] = dinv * h2


def _out_body(degA_ref, degB_ref, rA_ref, rB_ref, g2_ref, b2_ref, o_ref):
    dinv = _dinv_blk(degA_ref, degB_ref)
    o_ref[...] = dinv * (rA_ref[...] + rB_ref[...] + g2_ref[...]) + b2_ref[...]


def _row_spec(width):
    return pl.BlockSpec((ROW_BLK, width), lambda i: (i, 0))


def _full_spec(shape):
    return pl.BlockSpec(shape, lambda i: tuple(0 for _ in shape))


def kernel(x, edge_index, W1, b1, W2, b2):
    src = edge_index[0].astype(jnp.int32)
    dst = edge_index[1].astype(jnp.int32)
    pad = N_CHUNKS * CHUNK - EDGES_PER_TILE
    src_e = jnp.pad(src.reshape(N_TILES, EDGES_PER_TILE), ((0, 0), (0, pad)),
                    constant_values=0).reshape(2, 16, N_CHUNKS, CHUNK)
    padvals = N_NODES + jnp.arange(pad, dtype=jnp.int32) % (ACC_N - N_NODES)
    dst_e = jnp.concatenate(
        [dst.reshape(N_TILES, EDGES_PER_TILE),
         jnp.broadcast_to(padvals, (N_TILES, pad))],
        axis=1).reshape(2, 16, N_CHUNKS, CHUNK)
    x_pad = jnp.pad(x, ((0, NPAD - N_NODES), (0, 0)))

    deg = _make_deg(dst_e)
    degA, degB = deg[0], deg[1]

    h1 = pl.pallas_call(
        _mm1_body,
        grid=(_GRID,),
        in_specs=[_row_spec(D_IN), _full_spec((D_IN, D_HID))],
        out_specs=_row_spec(D_HID),
        out_shape=jax.ShapeDtypeStruct((NPAD, D_HID), jnp.float32),
    )(x_pad, W1)

    g1 = pl.pallas_call(
        _g1_body,
        grid=(_GRID,),
        in_specs=[_row_spec(DEG_W), _row_spec(DEG_W), _row_spec(D_HID)],
        out_specs=_row_spec(D_HID),
        out_shape=jax.ShapeDtypeStruct((NPAD, D_HID), jnp.float32),
    )(degA, degB, h1)

    r1 = _make_agg(src_e, dst_e, g1)

    W2p = jnp.pad(W2, ((0, 0), (0, D_HID - D_OUT)))
    b2p = jnp.pad(b2, (0, D_HID - D_OUT))

    g2 = pl.pallas_call(
        _mid_body,
        grid=(_GRID,),
        in_specs=[_row_spec(DEG_W), _row_spec(DEG_W), _row_spec(D_HID),
                  _row_spec(D_HID), _row_spec(D_HID),
                  _full_spec((1, D_HID)), _full_spec((D_HID, D_HID))],
        out_specs=_row_spec(D_HID),
        out_shape=jax.ShapeDtypeStruct((NPAD, D_HID), jnp.float32),
    )(degA, degB, r1[0], r1[1], g1, b1.reshape(1, D_HID), W2p)

    r2 = _make_agg(src_e, dst_e, g2)

    out = pl.pallas_call(
        _out_body,
        grid=(_GRID,),
        in_specs=[_row_spec(DEG_W), _row_spec(DEG_W), _row_spec(D_HID),
                  _row_spec(D_HID), _row_spec(D_HID), _full_spec((1, D_HID))],
        out_specs=_row_spec(D_HID),
        out_shape=jax.ShapeDtypeStruct((NPAD, D_HID), jnp.float32),
    )(degA, degB, r2[0], r2[1], g2, b2p.reshape(1, D_HID))

    return out[:N_NODES, :D_OUT]
